# Initial kernel scaffold; baseline (speedup 1.0000x reference)
import jax
import jax.numpy as jnp
from jax.experimental import pallas as pl


def kernel(edge_index, features, batch, emb, W1, b1, W2, b2, W3, b3, fcW, fcb):
    raise NotImplementedError("write your pallas kernel here")


# speedup vs baseline: 14.0928x; 14.0928x over previous
import functools

import jax
import jax.numpy as jnp
from jax import lax
from jax.experimental import pallas as pl
from jax.experimental.pallas import tpu as pltpu
import jax.experimental.pallas.tpu_sc as plsc

_N = 10000
_E = 320000
_G = 64
_D = 128
_VOCAB = 10000

_NC = 2
_NS = 16
_NW = _NC * _NS

_CH = 128
_CPW = 79
_EPW = _CPW * _CH
_EPAD = _NW * _EPW
_NPW = 320
_NPAD = _NW * _NPW
_FCH = 80
_NACC = 10112
_RPS = _NACC // _NS
_DPS = _NPAD // _NS


def _sc_deg_emb(dst_hbm, feat_hbm, emb_hbm, deg_hbm, x0_hbm,
                didx, fidx, ones, zb, rows, dacc, sem):
    c = lax.axis_index("c")
    s = lax.axis_index("s")
    wid = c * _NS + s

    def _init(i, _):
        ones[pl.ds(i * 16, 16)] = jnp.full((16,), 1.0, jnp.float32)
        zb[pl.ds(i * 16, 16)] = jnp.zeros((16,), jnp.float32)
        return 0
    lax.fori_loop(0, _DPS // 16, _init, 0)
    pltpu.sync_copy(zb, dacc.at[pl.ds(s * _DPS, _DPS)])
    plsc.subcore_barrier()

    ebase = wid * _EPW
    def _edge(j, _):
        pltpu.sync_copy(dst_hbm.at[pl.ds(ebase + j * _CH, _CH)], didx)
        pltpu.sync_copy(ones.at[pl.ds(0, _CH)], dacc.at[didx], add=True)
        return 0
    lax.fori_loop(0, _CPW, _edge, 0)

    fbase = wid * _NPW
    def _feat(j, _):
        off = fbase + j * _FCH
        pltpu.sync_copy(feat_hbm.at[pl.ds(off, _FCH)], fidx)
        pltpu.async_copy(emb_hbm.at[fidx], rows, sem).wait()
        pltpu.sync_copy(rows, x0_hbm.at[pl.ds(off, _FCH)])
        return 0
    lax.fori_loop(0, _NPW // _FCH, _feat, 0)

    plsc.subcore_barrier()
    pltpu.sync_copy(dacc.at[pl.ds(s * _DPS, _DPS)],
                    deg_hbm.at[pl.ds(c * _NPAD + s * _DPS, _DPS)])


@functools.lru_cache(maxsize=None)
def _deg_emb_call():
    mesh = plsc.VectorSubcoreMesh(core_axis_name="c", subcore_axis_name="s",
                                  num_cores=_NC, num_subcores=_NS)
    return pl.kernel(
        _sc_deg_emb,
        out_type=(jax.ShapeDtypeStruct((_NC * _NPAD,), jnp.float32),
                  jax.ShapeDtypeStruct((_NPAD, _D), jnp.float32)),
        mesh=mesh,
        scratch_types=[
            pltpu.VMEM((_CH,), jnp.int32),
            pltpu.VMEM((_FCH,), jnp.int32),
            pltpu.VMEM((_DPS,), jnp.float32),
            pltpu.VMEM((_DPS,), jnp.float32),
            pltpu.VMEM((_FCH, _D), jnp.float32),
            pltpu.VMEM_SHARED((_NPAD,), jnp.float32),
            pltpu.SemaphoreType.DMA,
        ],
    )


def _sc_agg(z_hbm, src_hbm, dst_hbm, out_hbm,
            sidx, didx, rows, zb, acc, sem):
    c = lax.axis_index("c")
    s = lax.axis_index("s")
    wid = c * _NS + s

    def _zb(i, _):
        for j in range(_D // 16):
            zb[i, pl.ds(j * 16, 16)] = jnp.zeros((16,), jnp.float32)
        return 0
    lax.fori_loop(0, 64, _zb, 0)
    rbase = s * _RPS
    for k in range(9):
        pltpu.sync_copy(zb, acc.at[pl.ds(rbase + k * 64, 64)])
    pltpu.sync_copy(zb.at[pl.ds(0, _RPS - 9 * 64)],
                    acc.at[pl.ds(rbase + 9 * 64, _RPS - 9 * 64)])
    plsc.subcore_barrier()

    ebase = wid * _EPW
    def _edge(j, _):
        off = ebase + j * _CH
        pltpu.sync_copy(src_hbm.at[pl.ds(off, _CH)], sidx)
        pltpu.async_copy(z_hbm.at[sidx], rows, sem).wait()
        pltpu.sync_copy(dst_hbm.at[pl.ds(off, _CH)], didx)
        pltpu.sync_copy(rows, acc.at[didx], add=True)
        return 0
    lax.fori_loop(0, _CPW, _edge, 0)

    plsc.subcore_barrier()
    pltpu.sync_copy(acc.at[pl.ds(rbase, _RPS)],
                    out_hbm.at[pl.ds(c * _NACC + rbase, _RPS)])


@functools.lru_cache(maxsize=None)
def _agg_call():
    mesh = plsc.VectorSubcoreMesh(core_axis_name="c", subcore_axis_name="s",
                                  num_cores=_NC, num_subcores=_NS)
    return pl.kernel(
        _sc_agg,
        out_type=jax.ShapeDtypeStruct((_NC * _NACC, _D), jnp.float32),
        mesh=mesh,
        scratch_types=[
            pltpu.VMEM((_CH,), jnp.int32),
            pltpu.VMEM((_CH,), jnp.int32),
            pltpu.VMEM((_CH, _D), jnp.float32),
            pltpu.VMEM((64, _D), jnp.float32),
            pltpu.VMEM_SHARED((_NACC, _D), jnp.float32),
            pltpu.SemaphoreType.DMA,
        ],
    )


_R = 2000
_NB = _N // _R


def _tc_first(d0, d1, x, w, dinv_o, z_o):
    deg = d0[...] + d1[...] + 1.0
    dv = lax.rsqrt(deg)
    dinv_o[...] = dv
    z_o[...] = jnp.dot(x[...] * dv, w[...], preferred_element_type=jnp.float32)


def _tc_mid(a0, a1, z, dv, b, w, z_o):
    x = jnp.maximum((a0[...] + a1[...] + z[...]) * dv[...] + b[...], 0.0)
    z_o[...] = jnp.dot(x * dv[...], w[...], preferred_element_type=jnp.float32)


def _tc_last(a0, a1, z, dv, b, bt, fcw, fcb, out, sums_s, cnt_s):
    i = pl.program_id(0)
    x = jnp.maximum((a0[...] + a1[...] + z[...]) * dv[...] + b[...], 0.0)
    brow = bt[0, 0, :]
    gids = lax.broadcasted_iota(jnp.int32, (_G, _R), 0)
    oh = jnp.where(brow[None, :] == gids, 1.0, 0.0)

    @pl.when(i == 0)
    def _():
        sums_s[...] = jnp.zeros_like(sums_s)
        cnt_s[...] = jnp.zeros_like(cnt_s)

    sums_s[...] += jnp.dot(oh, x, preferred_element_type=jnp.float32)
    cnt_s[...] += jnp.broadcast_to(jnp.sum(oh, axis=1, keepdims=True), (_G, _D))

    @pl.when(i == pl.num_programs(0) - 1)
    def _():
        pooled = sums_s[...] / jnp.maximum(cnt_s[...], 1.0)
        out[...] = jnp.dot(pooled, fcw[...],
                           preferred_element_type=jnp.float32) + fcb[...]


def _row_spec(cols):
    return pl.BlockSpec((_R, cols), lambda i: (i, 0))


def _full_spec(r, c):
    return pl.BlockSpec((r, c), lambda i: (0, 0))


_tc_first_call = pl.pallas_call(
    _tc_first,
    grid=(_NB,),
    in_specs=[_row_spec(1), _row_spec(1), _row_spec(_D), _full_spec(_D, _D)],
    out_specs=[_row_spec(1), _row_spec(_D)],
    out_shape=[jax.ShapeDtypeStruct((_N, 1), jnp.float32),
               jax.ShapeDtypeStruct((_N, _D), jnp.float32)],
)

_tc_mid_call = pl.pallas_call(
    _tc_mid,
    grid=(_NB,),
    in_specs=[_row_spec(_D), _row_spec(_D), _row_spec(_D), _row_spec(1),
              _full_spec(1, _D), _full_spec(_D, _D)],
    out_specs=_row_spec(_D),
    out_shape=jax.ShapeDtypeStruct((_N, _D), jnp.float32),
)

_tc_last_call = pl.pallas_call(
    _tc_last,
    grid=(_NB,),
    in_specs=[_row_spec(_D), _row_spec(_D), _row_spec(_D), _row_spec(1),
              _full_spec(1, _D),
              pl.BlockSpec((1, 1, _R), lambda i: (i, 0, 0)),
              _full_spec(_D, 1), _full_spec(1, 1)],
    out_specs=_full_spec(_G, 1),
    out_shape=jax.ShapeDtypeStruct((_G, 1), jnp.float32),
    scratch_shapes=[pltpu.VMEM((_G, _D), jnp.float32),
                    pltpu.VMEM((_G, _D), jnp.float32)],
)


def kernel(edge_index, features, batch, emb, W1, b1, W2, b2, W3, b3, fcW, fcb):
    f32 = jnp.float32
    npad = _EPAD - _E
    pad_ids = jnp.arange(npad, dtype=jnp.int32)
    srcp = jnp.concatenate([edge_index[0], (pad_ids * 997) % _N])
    dstp = jnp.concatenate([edge_index[1], _N + (pad_ids % 16)])
    featp = jnp.concatenate(
        [features[:, 0], (jnp.arange(_NPAD - _N, dtype=jnp.int32) * 131) % _VOCAB])

    deg_part, x0p = _deg_emb_call()(dstp, featp, emb)
    x0 = x0p[:_N]
    deg0 = deg_part[:_N, None]
    deg1 = deg_part[_NPAD:_NPAD + _N, None]

    dinv, z0 = _tc_first_call(deg0, deg1, x0, W1)

    b1r = b1[None, :]
    b2r = b2[None, :]
    b3r = b3[None, :]
    bt = batch.reshape(_NB, 1, _R)

    s0 = _agg_call()(z0, srcp, dstp)
    z1 = _tc_mid_call(s0[:_N], s0[_NACC:_NACC + _N], z0, dinv, b1r, W2)
    s1 = _agg_call()(z1, srcp, dstp)
    z2 = _tc_mid_call(s1[:_N], s1[_NACC:_NACC + _N], z1, dinv, b2r, W3)
    s2 = _agg_call()(z2, srcp, dstp)
    out = _tc_last_call(s2[:_N], s2[_NACC:_NACC + _N], z2, dinv, b3r, bt,
                        fcW, fcb.reshape(1, 1))
    return out[:, 0].astype(f32)


# speedup vs baseline: 28.0176x; 1.9881x over previous
import functools

import jax
import jax.numpy as jnp
from jax import lax
from jax.experimental import pallas as pl
from jax.experimental.pallas import tpu as pltpu
import jax.experimental.pallas.tpu_sc as plsc

_N = 10000
_E = 320000
_G = 64
_D = 128
_VOCAB = 10000

_NC = 2
_NS = 16
_NW = _NC * _NS

_CH = 128
_CPW = 80
_CPWI = _CPW + 4
_EPW = _CPW * _CH
_EPAD = _NW * _EPW
_NPW = 320
_NPAD = _NW * _NPW
_FCH = 80
_NACC = 10112
_RPS = _NACC // _NS
_DPS = _NPAD // _NS


def _sc_deg_emb(dst_hbm, feat_hbm, emb_hbm, deg_hbm, x0_hbm,
                didx2, fidx, ones, zb, rows, dacc, sem):
    c = lax.axis_index("c")
    s = lax.axis_index("s")
    wid = c * _NS + s

    def _init(i, _):
        ones[pl.ds(i * 16, 16)] = jnp.full((16,), 1.0, jnp.float32)
        zb[pl.ds(i * 16, 16)] = jnp.zeros((16,), jnp.float32)
        return 0
    lax.fori_loop(0, _DPS // 16, _init, 0)
    pltpu.sync_copy(zb, dacc.at[pl.ds(s * _DPS, _DPS)])
    pltpu.sync_copy(dst_hbm.at[wid], didx2)
    plsc.subcore_barrier()

    def _edge(j, _):
        pltpu.sync_copy(ones.at[pl.ds(0, _CH)], dacc.at[didx2.at[j]], add=True)
        return 0
    lax.fori_loop(0, _CPW, _edge, 0)

    fbase = wid * _NPW
    def _feat(j, _):
        off = fbase + j * _FCH
        pltpu.sync_copy(feat_hbm.at[pl.ds(off, _FCH)], fidx)
        pltpu.async_copy(emb_hbm.at[fidx], rows, sem).wait()
        pltpu.sync_copy(rows, x0_hbm.at[pl.ds(off, _FCH)])
        return 0
    lax.fori_loop(0, _NPW // _FCH, _feat, 0)

    plsc.subcore_barrier()
    pltpu.sync_copy(dacc.at[pl.ds(s * _DPS, _DPS)],
                    deg_hbm.at[pl.ds(c * _NPAD + s * _DPS, _DPS)])


@functools.lru_cache(maxsize=None)
def _deg_emb_call():
    mesh = plsc.VectorSubcoreMesh(core_axis_name="c", subcore_axis_name="s",
                                  num_cores=_NC, num_subcores=_NS)
    return pl.kernel(
        _sc_deg_emb,
        out_type=(jax.ShapeDtypeStruct((_NC * _NPAD,), jnp.float32),
                  jax.ShapeDtypeStruct((_NPAD, _D), jnp.float32)),
        mesh=mesh,
        scratch_types=[
            pltpu.VMEM((_CPWI, _CH), jnp.int32),
            pltpu.VMEM((_FCH,), jnp.int32),
            pltpu.VMEM((_DPS,), jnp.float32),
            pltpu.VMEM((_DPS,), jnp.float32),
            pltpu.VMEM((_FCH, _D), jnp.float32),
            pltpu.VMEM_SHARED((_NPAD,), jnp.float32),
            pltpu.SemaphoreType.DMA,
        ],
    )


def _sc_agg(z_hbm, src_hbm, dst_hbm, out_hbm,
            sidx2, dring, rows, acc,
            gsem0, gsem1, isem0, isem1, isem2, isem3):
    c = lax.axis_index("c")
    s = lax.axis_index("s")
    wid = c * _NS + s
    rbase = s * _RPS

    def _zr(i, _):
        for j in range(_D // 16):
            rows[0, i, pl.ds(j * 16, 16)] = jnp.zeros((16,), jnp.float32)
        return 0
    lax.fori_loop(0, _CH, _zr, 0)
    for k in range(4):
        pltpu.sync_copy(rows.at[0], acc.at[pl.ds(rbase + k * _CH, _CH)])
    pltpu.sync_copy(rows.at[0, pl.ds(0, _RPS - 4 * _CH)],
                    acc.at[pl.ds(rbase + 4 * _CH, _RPS - 4 * _CH)])
    pltpu.sync_copy(src_hbm.at[wid], sidx2)
    plsc.subcore_barrier()

    gsems = (gsem0, gsem1)
    isems = (isem0, isem1, isem2, isem3)
    for k in range(4):
        pltpu.async_copy(dst_hbm.at[wid, k], dring.at[k], isems[k])
    for b in range(2):
        pltpu.async_copy(z_hbm.at[sidx2.at[b]], rows.at[b], gsems[b])

    def _quad(jq, _):
        for b in range(4):
            j = jq * 4 + b
            buf = b % 2
            pltpu.make_async_copy(z_hbm.at[sidx2.at[j]], rows.at[buf],
                                  gsems[buf]).wait()
            pltpu.make_async_copy(dst_hbm.at[wid, j], dring.at[b],
                                  isems[b]).wait()
            pltpu.sync_copy(rows.at[buf], acc.at[dring.at[b]], add=True)
            pltpu.async_copy(dst_hbm.at[wid, j + 4], dring.at[b], isems[b])
            pltpu.async_copy(z_hbm.at[sidx2.at[j + 2]], rows.at[buf],
                             gsems[buf])
        return 0
    lax.fori_loop(0, _CPW // 4, _quad, 0)
    for b in range(2):
        pltpu.make_async_copy(z_hbm.at[sidx2.at[_CPW + b]], rows.at[b],
                              gsems[b]).wait()
    for b in range(4):
        pltpu.make_async_copy(dst_hbm.at[wid, _CPW + b], dring.at[b],
                              isems[b]).wait()

    plsc.subcore_barrier()
    pltpu.sync_copy(acc.at[pl.ds(rbase, _RPS)],
                    out_hbm.at[pl.ds(c * _NACC + rbase, _RPS)])


@functools.lru_cache(maxsize=None)
def _agg_call():
    mesh = plsc.VectorSubcoreMesh(core_axis_name="c", subcore_axis_name="s",
                                  num_cores=_NC, num_subcores=_NS)
    return pl.kernel(
        _sc_agg,
        out_type=jax.ShapeDtypeStruct((_NC * _NACC, _D), jnp.float32),
        mesh=mesh,
        scratch_types=[
            pltpu.VMEM((_CPWI, _CH), jnp.int32),
            pltpu.VMEM((4, _CH), jnp.int32),
            pltpu.VMEM((2, _CH, _D), jnp.float32),
            pltpu.VMEM_SHARED((_NACC, _D), jnp.float32),
            pltpu.SemaphoreType.DMA,
            pltpu.SemaphoreType.DMA,
            pltpu.SemaphoreType.DMA,
            pltpu.SemaphoreType.DMA,
            pltpu.SemaphoreType.DMA,
            pltpu.SemaphoreType.DMA,
        ],
    )


_R = 2000
_NB = _N // _R


def _tc_first(d0, d1, x, w, dinv_o, z_o):
    deg = d0[...] + d1[...] + 1.0
    dv = lax.rsqrt(deg)
    dinv_o[...] = dv
    z_o[...] = jnp.dot(x[...] * dv, w[...], preferred_element_type=jnp.float32)


def _tc_mid(a0, a1, z, dv, b, w, z_o):
    x = jnp.maximum((a0[...] + a1[...] + z[...]) * dv[...] + b[...], 0.0)
    z_o[...] = jnp.dot(x * dv[...], w[...], preferred_element_type=jnp.float32)


def _tc_last(a0, a1, z, dv, b, bt, fcw, fcb, out, sums_s, cnt_s):
    i = pl.program_id(0)
    x = jnp.maximum((a0[...] + a1[...] + z[...]) * dv[...] + b[...], 0.0)
    brow = bt[0, 0, :]
    gids = lax.broadcasted_iota(jnp.int32, (_G, _R), 0)
    oh = jnp.where(brow[None, :] == gids, 1.0, 0.0)

    @pl.when(i == 0)
    def _():
        sums_s[...] = jnp.zeros_like(sums_s)
        cnt_s[...] = jnp.zeros_like(cnt_s)

    sums_s[...] += jnp.dot(oh, x, preferred_element_type=jnp.float32)
    cnt_s[...] += jnp.broadcast_to(jnp.sum(oh, axis=1, keepdims=True), (_G, _D))

    @pl.when(i == pl.num_programs(0) - 1)
    def _():
        pooled = sums_s[...] / jnp.maximum(cnt_s[...], 1.0)
        out[...] = jnp.dot(pooled, fcw[...],
                           preferred_element_type=jnp.float32) + fcb[...]


def _row_spec(cols):
    return pl.BlockSpec((_R, cols), lambda i: (i, 0))


def _full_spec(r, c):
    return pl.BlockSpec((r, c), lambda i: (0, 0))


_tc_first_call = pl.pallas_call(
    _tc_first,
    grid=(_NB,),
    in_specs=[_row_spec(1), _row_spec(1), _row_spec(_D), _full_spec(_D, _D)],
    out_specs=[_row_spec(1), _row_spec(_D)],
    out_shape=[jax.ShapeDtypeStruct((_N, 1), jnp.float32),
               jax.ShapeDtypeStruct((_N, _D), jnp.float32)],
)

_tc_mid_call = pl.pallas_call(
    _tc_mid,
    grid=(_NB,),
    in_specs=[_row_spec(_D), _row_spec(_D), _row_spec(_D), _row_spec(1),
              _full_spec(1, _D), _full_spec(_D, _D)],
    out_specs=_row_spec(_D),
    out_shape=jax.ShapeDtypeStruct((_N, _D), jnp.float32),
)

_tc_last_call = pl.pallas_call(
    _tc_last,
    grid=(_NB,),
    in_specs=[_row_spec(_D), _row_spec(_D), _row_spec(_D), _row_spec(1),
              _full_spec(1, _D),
              pl.BlockSpec((1, 1, _R), lambda i: (i, 0, 0)),
              _full_spec(_D, 1), _full_spec(1, 1)],
    out_specs=_full_spec(_G, 1),
    out_shape=jax.ShapeDtypeStruct((_G, 1), jnp.float32),
    scratch_shapes=[pltpu.VMEM((_G, _D), jnp.float32),
                    pltpu.VMEM((_G, _D), jnp.float32)],
)


def kernel(edge_index, features, batch, emb, W1, b1, W2, b2, W3, b3, fcW, fcb):
    f32 = jnp.float32
    npad = _EPAD - _E
    pad_ids = jnp.arange(npad, dtype=jnp.int32)
    dum_ids = jnp.arange(_NW * 4 * _CH, dtype=jnp.int32)
    srcp = jnp.concatenate([edge_index[0], (pad_ids * 997) % _N])
    dstp = jnp.concatenate([edge_index[1], _N + (pad_ids % (_NACC - _N))])
    src3 = jnp.concatenate(
        [srcp.reshape(_NW, _EPW),
         ((dum_ids * 37) % _N).reshape(_NW, 4 * _CH)], axis=1
    ).reshape(_NW, _CPWI, _CH)
    dst3 = jnp.concatenate(
        [dstp.reshape(_NW, _EPW),
         (_N + dum_ids % (_NACC - _N)).reshape(_NW, 4 * _CH)], axis=1
    ).reshape(_NW, _CPWI, _CH)
    featp = jnp.concatenate(
        [features[:, 0], (jnp.arange(_NPAD - _N, dtype=jnp.int32) * 131) % _VOCAB])

    deg_part, x0p = _deg_emb_call()(dst3, featp, emb)
    x0 = x0p[:_N]
    deg0 = deg_part[:_N, None]
    deg1 = deg_part[_NPAD:_NPAD + _N, None]

    dinv, z0 = _tc_first_call(deg0, deg1, x0, W1)

    b1r = b1[None, :]
    b2r = b2[None, :]
    b3r = b3[None, :]
    bt = batch.reshape(_NB, 1, _R)

    s0 = _agg_call()(z0, src3, dst3)
    z1 = _tc_mid_call(s0[:_N], s0[_NACC:_NACC + _N], z0, dinv, b1r, W2)
    s1 = _agg_call()(z1, src3, dst3)
    z2 = _tc_mid_call(s1[:_N], s1[_NACC:_NACC + _N], z1, dinv, b2r, W3)
    s2 = _agg_call()(z2, src3, dst3)
    out = _tc_last_call(s2[:_N], s2[_NACC:_NACC + _N], z2, dinv, b3r, bt,
                        fcW, fcb.reshape(1, 1))
    return out[:, 0].astype(f32)


# speedup vs baseline: 28.6900x; 1.0240x over previous
import functools

import jax
import jax.numpy as jnp
from jax import lax
from jax.experimental import pallas as pl
from jax.experimental.pallas import tpu as pltpu
import jax.experimental.pallas.tpu_sc as plsc

_N = 10000
_E = 320000
_G = 64
_D = 128
_VOCAB = 10000

_NC = 2
_NS = 16
_NW = _NC * _NS

_CH = 128
_CPW = 80
_EPW = _CPW * _CH
_EPAD = _NW * _EPW
_NPW = 320
_NPAD = _NW * _NPW
_FCH = 80
_NACC = 10112
_RPS = _NACC // _NS
_DPS = _NPAD // _NS


def _sc_deg_emb(dst_hbm, feat_hbm, emb_hbm, deg_hbm, x0_hbm,
                didx2, fidx, ones, zb, rows2, dacc, sem, fsem0, fsem1):
    c = lax.axis_index("c")
    s = lax.axis_index("s")
    wid = c * _NS + s

    def _init(i, _):
        ones[pl.ds(i * 16, 16)] = jnp.full((16,), 1.0, jnp.float32)
        zb[pl.ds(i * 16, 16)] = jnp.zeros((16,), jnp.float32)
        return 0
    lax.fori_loop(0, _DPS // 16, _init, 0)
    pltpu.sync_copy(zb, dacc.at[pl.ds(s * _DPS, _DPS)])
    pltpu.sync_copy(dst_hbm.at[wid], didx2)
    plsc.subcore_barrier()

    def _grp(g, _):
        for k in range(8):
            pltpu.async_copy(ones.at[pl.ds(0, _CH)],
                             dacc.at[didx2.at[g * 8 + k]], sem, add=True)
        for k in range(8):
            pltpu.make_async_copy(ones.at[pl.ds(0, _CH)],
                                  dacc.at[didx2.at[g * 8 + k]], sem).wait()
        return 0
    lax.fori_loop(0, _CPW // 8, _grp, 0)

    fbase = wid * _NPW
    fsems = (fsem0, fsem1)
    for j in range(_NPW // _FCH):
        jb = j % 2
        pltpu.sync_copy(feat_hbm.at[pl.ds(fbase + j * _FCH, _FCH)],
                        fidx.at[jb])
        pltpu.async_copy(emb_hbm.at[fidx.at[jb]], rows2.at[jb], fsems[jb])
        if j > 0:
            pb = (j - 1) % 2
            off = fbase + (j - 1) * _FCH
            pltpu.make_async_copy(emb_hbm.at[fidx.at[pb]], rows2.at[pb],
                                  fsems[pb]).wait()
            pltpu.sync_copy(rows2.at[pb], x0_hbm.at[pl.ds(off, _FCH)])
    lastb = (_NPW // _FCH - 1) % 2
    lasto = fbase + (_NPW - _FCH)
    pltpu.make_async_copy(emb_hbm.at[fidx.at[lastb]], rows2.at[lastb],
                          fsems[lastb]).wait()
    pltpu.sync_copy(rows2.at[lastb], x0_hbm.at[pl.ds(lasto, _FCH)])

    plsc.subcore_barrier()
    pltpu.sync_copy(dacc.at[pl.ds(s * _DPS, _DPS)],
                    deg_hbm.at[pl.ds(c * _NPAD + s * _DPS, _DPS)])


@functools.lru_cache(maxsize=None)
def _deg_emb_call():
    mesh = plsc.VectorSubcoreMesh(core_axis_name="c", subcore_axis_name="s",
                                  num_cores=_NC, num_subcores=_NS)
    return pl.kernel(
        _sc_deg_emb,
        out_type=(jax.ShapeDtypeStruct((_NC * _NPAD,), jnp.float32),
                  jax.ShapeDtypeStruct((_NPAD, _D), jnp.float32)),
        mesh=mesh,
        scratch_types=[
            pltpu.VMEM((_CPW, _CH), jnp.int32),
            pltpu.VMEM((2, _FCH), jnp.int32),
            pltpu.VMEM((_DPS,), jnp.float32),
            pltpu.VMEM((_DPS,), jnp.float32),
            pltpu.VMEM((2, _FCH, _D), jnp.float32),
            pltpu.VMEM_SHARED((_NPAD,), jnp.float32),
            pltpu.SemaphoreType.DMA,
            pltpu.SemaphoreType.DMA,
            pltpu.SemaphoreType.DMA,
        ],
    )


def _sc_agg(z_hbm, src_hbm, dst_hbm, out_hbm,
            sidx2, dring, rows, acc,
            gsem0, gsem1, isem0, isem1, isem2, isem3):
    c = lax.axis_index("c")
    s = lax.axis_index("s")
    wid = c * _NS + s
    rbase = s * _RPS

    def _zr(i, _):
        for j in range(_D // 16):
            rows[0, i, pl.ds(j * 16, 16)] = jnp.zeros((16,), jnp.float32)
        return 0
    lax.fori_loop(0, _CH, _zr, 0)
    for k in range(4):
        pltpu.sync_copy(rows.at[0], acc.at[pl.ds(rbase + k * _CH, _CH)])
    pltpu.sync_copy(rows.at[0, pl.ds(0, _RPS - 4 * _CH)],
                    acc.at[pl.ds(rbase + 4 * _CH, _RPS - 4 * _CH)])
    pltpu.sync_copy(src_hbm.at[wid], sidx2)
    plsc.subcore_barrier()

    gsems = (gsem0, gsem1)
    isems = (isem0, isem1, isem2, isem3)
    for k in range(4):
        pltpu.async_copy(dst_hbm.at[wid, k], dring.at[k], isems[k])
    for b in range(2):
        pltpu.async_copy(z_hbm.at[sidx2.at[b]], rows.at[b], gsems[b])

    def _quad(jq, _):
        for b in range(4):
            j = jq * 4 + b
            buf = b % 2
            pltpu.make_async_copy(z_hbm.at[sidx2.at[j]], rows.at[buf],
                                  gsems[buf]).wait()
            pltpu.make_async_copy(dst_hbm.at[wid, j], dring.at[b],
                                  isems[b]).wait()
            pltpu.sync_copy(rows.at[buf], acc.at[dring.at[b]], add=True)
            pltpu.async_copy(dst_hbm.at[wid, j + 4], dring.at[b], isems[b])
            pltpu.async_copy(z_hbm.at[sidx2.at[j + 2]], rows.at[buf],
                             gsems[buf])
        return 0
    lax.fori_loop(0, _CPW // 4 - 1, _quad, 0)
    for b in range(4):
        j = _CPW - 4 + b
        buf = b % 2
        pltpu.make_async_copy(z_hbm.at[sidx2.at[j]], rows.at[buf],
                              gsems[buf]).wait()
        pltpu.make_async_copy(dst_hbm.at[wid, j], dring.at[b],
                              isems[b]).wait()
        pltpu.sync_copy(rows.at[buf], acc.at[dring.at[b]], add=True)
        if j + 2 < _CPW:
            pltpu.async_copy(z_hbm.at[sidx2.at[j + 2]], rows.at[buf],
                             gsems[buf])

    plsc.subcore_barrier()
    nlast = _N - (_NS - 1) * _RPS
    @pl.when(s < _NS - 1)
    def _():
        pltpu.sync_copy(acc.at[pl.ds(rbase, _RPS)],
                        out_hbm.at[c, pl.ds(rbase, _RPS)])
    @pl.when(s == _NS - 1)
    def _():
        pltpu.sync_copy(acc.at[pl.ds(rbase, nlast)],
                        out_hbm.at[c, pl.ds(rbase, nlast)])


@functools.lru_cache(maxsize=None)
def _agg_call():
    mesh = plsc.VectorSubcoreMesh(core_axis_name="c", subcore_axis_name="s",
                                  num_cores=_NC, num_subcores=_NS)
    return pl.kernel(
        _sc_agg,
        out_type=jax.ShapeDtypeStruct((_NC, _N, _D), jnp.float32),
        mesh=mesh,
        scratch_types=[
            pltpu.VMEM((_CPW, _CH), jnp.int32),
            pltpu.VMEM((4, _CH), jnp.int32),
            pltpu.VMEM((2, _CH, _D), jnp.float32),
            pltpu.VMEM_SHARED((_NACC, _D), jnp.float32),
            pltpu.SemaphoreType.DMA,
            pltpu.SemaphoreType.DMA,
            pltpu.SemaphoreType.DMA,
            pltpu.SemaphoreType.DMA,
            pltpu.SemaphoreType.DMA,
            pltpu.SemaphoreType.DMA,
        ],
    )


_R = 2000
_NB = _N // _R


def _tc_first(d0, d1, x, w, dinv_o, z_o):
    deg = d0[...] + d1[...] + 1.0
    dv = lax.rsqrt(deg)
    dinv_o[...] = dv
    z_o[...] = jnp.dot(x[...] * dv, w[...], preferred_element_type=jnp.float32)


def _tc_mid(a0, a1, z, dv, b, w, z_o):
    x = jnp.maximum((a0[...] + a1[...] + z[...]) * dv[...] + b[...], 0.0)
    z_o[...] = jnp.dot(x * dv[...], w[...], preferred_element_type=jnp.float32)


def _tc_last(a0, a1, z, dv, b, bt, fcw, fcb, out, sums_s, cnt_s):
    i = pl.program_id(0)
    x = jnp.maximum((a0[...] + a1[...] + z[...]) * dv[...] + b[...], 0.0)
    brow = bt[0, 0, :]
    gids = lax.broadcasted_iota(jnp.int32, (_G, _R), 0)
    oh = jnp.where(brow[None, :] == gids, 1.0, 0.0)

    @pl.when(i == 0)
    def _():
        sums_s[...] = jnp.zeros_like(sums_s)
        cnt_s[...] = jnp.zeros_like(cnt_s)

    sums_s[...] += jnp.dot(oh, x, preferred_element_type=jnp.float32)
    cnt_s[...] += jnp.broadcast_to(jnp.sum(oh, axis=1, keepdims=True), (_G, _D))

    @pl.when(i == pl.num_programs(0) - 1)
    def _():
        pooled = sums_s[...] / jnp.maximum(cnt_s[...], 1.0)
        out[...] = jnp.dot(pooled, fcw[...],
                           preferred_element_type=jnp.float32) + fcb[...]


def _row_spec(cols):
    return pl.BlockSpec((_R, cols), lambda i: (i, 0))


def _full_spec(r, c):
    return pl.BlockSpec((r, c), lambda i: (0, 0))


_tc_first_call = pl.pallas_call(
    _tc_first,
    grid=(_NB,),
    in_specs=[_row_spec(1), _row_spec(1), _row_spec(_D), _full_spec(_D, _D)],
    out_specs=[_row_spec(1), _row_spec(_D)],
    out_shape=[jax.ShapeDtypeStruct((_N, 1), jnp.float32),
               jax.ShapeDtypeStruct((_N, _D), jnp.float32)],
)

_tc_mid_call = pl.pallas_call(
    _tc_mid,
    grid=(_NB,),
    in_specs=[_row_spec(_D), _row_spec(_D), _row_spec(_D), _row_spec(1),
              _full_spec(1, _D), _full_spec(_D, _D)],
    out_specs=_row_spec(_D),
    out_shape=jax.ShapeDtypeStruct((_N, _D), jnp.float32),
)

_tc_last_call = pl.pallas_call(
    _tc_last,
    grid=(_NB,),
    in_specs=[_row_spec(_D), _row_spec(_D), _row_spec(_D), _row_spec(1),
              _full_spec(1, _D),
              pl.BlockSpec((1, 1, _R), lambda i: (i, 0, 0)),
              _full_spec(_D, 1), _full_spec(1, 1)],
    out_specs=_full_spec(_G, 1),
    out_shape=jax.ShapeDtypeStruct((_G, 1), jnp.float32),
    scratch_shapes=[pltpu.VMEM((_G, _D), jnp.float32),
                    pltpu.VMEM((_G, _D), jnp.float32)],
)


def kernel(edge_index, features, batch, emb, W1, b1, W2, b2, W3, b3, fcW, fcb):
    f32 = jnp.float32
    npad = _EPAD - _E
    pad_ids = jnp.arange(npad, dtype=jnp.int32)
    srcp = jnp.concatenate([edge_index[0], (pad_ids * 997) % _N])
    dstp = jnp.concatenate([edge_index[1], _N + (pad_ids % (_NACC - _N))])
    src3 = srcp.reshape(_NW, _CPW, _CH)
    dst3 = dstp.reshape(_NW, _CPW, _CH)
    featp = jnp.concatenate(
        [features[:, 0], (jnp.arange(_NPAD - _N, dtype=jnp.int32) * 131) % _VOCAB])

    deg_part, x0p = _deg_emb_call()(dst3, featp, emb)
    x0 = x0p[:_N]
    deg0 = deg_part[:_N, None]
    deg1 = deg_part[_NPAD:_NPAD + _N, None]

    dinv, z0 = _tc_first_call(deg0, deg1, x0, W1)

    b1r = b1[None, :]
    b2r = b2[None, :]
    b3r = b3[None, :]
    bt = batch.reshape(_NB, 1, _R)

    s0 = _agg_call()(z0, src3, dst3)
    z1 = _tc_mid_call(s0[0], s0[1], z0, dinv, b1r, W2)
    s1 = _agg_call()(z1, src3, dst3)
    z2 = _tc_mid_call(s1[0], s1[1], z1, dinv, b2r, W3)
    s2 = _agg_call()(z2, src3, dst3)
    out = _tc_last_call(s2[0], s2[1], z2, dinv, b3r, bt,
                        fcW, fcb.reshape(1, 1))
    return out[:, 0].astype(f32)


# speedup vs baseline: 29.0251x; 1.0117x over previous
import functools

import jax
import jax.numpy as jnp
from jax import lax
from jax.experimental import pallas as pl
from jax.experimental.pallas import tpu as pltpu
import jax.experimental.pallas.tpu_sc as plsc

_N = 10000
_E = 320000
_G = 64
_D = 128
_VOCAB = 10000

_NC = 2
_NS = 16
_NW = _NC * _NS

_CH = 128
_CPW = 80
_EPW = _CPW * _CH
_EPAD = _NW * _EPW
_NPW = 320
_NPAD = _NW * _NPW
_FCH = 80
_NACC = 10112
_RPS = _NACC // _NS
_DPS = _NPAD // _NS


def _sc_deg_emb(dst_hbm, feat_hbm, emb_hbm, deg_hbm, x0_hbm,
                didx2, fidx, ones, zb, rows2, dacc, sem, fsem0, fsem1):
    c = lax.axis_index("c")
    s = lax.axis_index("s")
    wid = c * _NS + s

    def _init(i, _):
        ones[pl.ds(i * 16, 16)] = jnp.full((16,), 1.0, jnp.float32)
        zb[pl.ds(i * 16, 16)] = jnp.zeros((16,), jnp.float32)
        return 0
    lax.fori_loop(0, _DPS // 16, _init, 0)
    pltpu.sync_copy(zb, dacc.at[pl.ds(s * _DPS, _DPS)])
    pltpu.sync_copy(dst_hbm.at[wid], didx2)
    plsc.subcore_barrier()

    def _grp(g, _):
        for k in range(8):
            pltpu.async_copy(ones.at[pl.ds(0, _CH)],
                             dacc.at[didx2.at[g * 8 + k]], sem, add=True)
        for k in range(8):
            pltpu.make_async_copy(ones.at[pl.ds(0, _CH)],
                                  dacc.at[didx2.at[g * 8 + k]], sem).wait()
        return 0
    lax.fori_loop(0, _CPW // 8, _grp, 0)

    fbase = wid * _NPW
    fsems = (fsem0, fsem1)
    for j in range(_NPW // _FCH):
        jb = j % 2
        pltpu.sync_copy(feat_hbm.at[pl.ds(fbase + j * _FCH, _FCH)],
                        fidx.at[jb])
        pltpu.async_copy(emb_hbm.at[fidx.at[jb]], rows2.at[jb], fsems[jb])
        if j > 0:
            pb = (j - 1) % 2
            off = fbase + (j - 1) * _FCH
            pltpu.make_async_copy(emb_hbm.at[fidx.at[pb]], rows2.at[pb],
                                  fsems[pb]).wait()
            pltpu.sync_copy(rows2.at[pb], x0_hbm.at[pl.ds(off, _FCH)])
    lastb = (_NPW // _FCH - 1) % 2
    lasto = fbase + (_NPW - _FCH)
    pltpu.make_async_copy(emb_hbm.at[fidx.at[lastb]], rows2.at[lastb],
                          fsems[lastb]).wait()
    pltpu.sync_copy(rows2.at[lastb], x0_hbm.at[pl.ds(lasto, _FCH)])

    plsc.subcore_barrier()
    pltpu.sync_copy(dacc.at[pl.ds(s * _DPS, _DPS)],
                    deg_hbm.at[pl.ds(c * _NPAD + s * _DPS, _DPS)])


@functools.lru_cache(maxsize=None)
def _deg_emb_call():
    mesh = plsc.VectorSubcoreMesh(core_axis_name="c", subcore_axis_name="s",
                                  num_cores=_NC, num_subcores=_NS)
    return pl.kernel(
        _sc_deg_emb,
        out_type=(jax.ShapeDtypeStruct((_NC * _NPAD,), jnp.float32),
                  jax.ShapeDtypeStruct((_NPAD, _D), jnp.float32)),
        mesh=mesh,
        scratch_types=[
            pltpu.VMEM((_CPW, _CH), jnp.int32),
            pltpu.VMEM((2, _FCH), jnp.int32),
            pltpu.VMEM((_DPS,), jnp.float32),
            pltpu.VMEM((_DPS,), jnp.float32),
            pltpu.VMEM((2, _FCH, _D), jnp.float32),
            pltpu.VMEM_SHARED((_NPAD,), jnp.float32),
            pltpu.SemaphoreType.DMA,
            pltpu.SemaphoreType.DMA,
            pltpu.SemaphoreType.DMA,
        ],
    )


def _sc_agg(z_hbm, src_hbm, dst_hbm, out_hbm,
            sidx2, dring, rows, acc,
            gsem0, gsem1, isem0, isem1, isem2, isem3, zsem):
    c = lax.axis_index("c")
    s = lax.axis_index("s")
    wid = c * _NS + s
    rbase = s * _RPS

    gsems = (gsem0, gsem1)
    isems = (isem0, isem1, isem2, isem3)

    def _zr(i, _):
        for j in range(_D // 16):
            rows[1, i, pl.ds(j * 16, 16)] = jnp.zeros((16,), jnp.float32)
        return 0
    lax.fori_loop(0, _CH, _zr, 0)
    pltpu.sync_copy(src_hbm.at[wid], sidx2)
    pltpu.async_copy(z_hbm.at[sidx2.at[0]], rows.at[0], gsem0)
    for k in range(4):
        pltpu.async_copy(dst_hbm.at[wid, k], dring.at[k], isems[k])
    for k in range(4):
        pltpu.async_copy(rows.at[1], acc.at[pl.ds(rbase + k * _CH, _CH)],
                         zsem)
    pltpu.async_copy(rows.at[1, pl.ds(0, _RPS - 4 * _CH)],
                     acc.at[pl.ds(rbase + 4 * _CH, _RPS - 4 * _CH)], zsem)
    for k in range(4):
        pltpu.make_async_copy(rows.at[1], acc.at[pl.ds(rbase + k * _CH, _CH)],
                              zsem).wait()
    pltpu.make_async_copy(rows.at[1, pl.ds(0, _RPS - 4 * _CH)],
                          acc.at[pl.ds(rbase + 4 * _CH, _RPS - 4 * _CH)],
                          zsem).wait()
    plsc.subcore_barrier()

    pltpu.async_copy(z_hbm.at[sidx2.at[1]], rows.at[1], gsem1)

    def _quad(jq, _):
        for b in range(4):
            j = jq * 4 + b
            buf = b % 2
            pltpu.make_async_copy(z_hbm.at[sidx2.at[j]], rows.at[buf],
                                  gsems[buf]).wait()
            pltpu.make_async_copy(dst_hbm.at[wid, j], dring.at[b],
                                  isems[b]).wait()
            pltpu.sync_copy(rows.at[buf], acc.at[dring.at[b]], add=True)
            pltpu.async_copy(dst_hbm.at[wid, j + 4], dring.at[b], isems[b])
            pltpu.async_copy(z_hbm.at[sidx2.at[j + 2]], rows.at[buf],
                             gsems[buf])
        return 0
    lax.fori_loop(0, _CPW // 4 - 1, _quad, 0)
    for b in range(4):
        j = _CPW - 4 + b
        buf = b % 2
        pltpu.make_async_copy(z_hbm.at[sidx2.at[j]], rows.at[buf],
                              gsems[buf]).wait()
        pltpu.make_async_copy(dst_hbm.at[wid, j], dring.at[b],
                              isems[b]).wait()
        pltpu.sync_copy(rows.at[buf], acc.at[dring.at[b]], add=True)
        if j + 2 < _CPW:
            pltpu.async_copy(z_hbm.at[sidx2.at[j + 2]], rows.at[buf],
                             gsems[buf])

    plsc.subcore_barrier()
    nlast = _N - (_NS - 1) * _RPS
    @pl.when(s < _NS - 1)
    def _():
        pltpu.sync_copy(acc.at[pl.ds(rbase, _RPS)],
                        out_hbm.at[c, pl.ds(rbase, _RPS)])
    @pl.when(s == _NS - 1)
    def _():
        pltpu.sync_copy(acc.at[pl.ds(rbase, nlast)],
                        out_hbm.at[c, pl.ds(rbase, nlast)])


@functools.lru_cache(maxsize=None)
def _agg_call():
    mesh = plsc.VectorSubcoreMesh(core_axis_name="c", subcore_axis_name="s",
                                  num_cores=_NC, num_subcores=_NS)
    return pl.kernel(
        _sc_agg,
        out_type=jax.ShapeDtypeStruct((_NC, _N, _D), jnp.float32),
        mesh=mesh,
        scratch_types=[
            pltpu.VMEM((_CPW, _CH), jnp.int32),
            pltpu.VMEM((4, _CH), jnp.int32),
            pltpu.VMEM((2, _CH, _D), jnp.float32),
            pltpu.VMEM_SHARED((_NACC, _D), jnp.float32),
            pltpu.SemaphoreType.DMA,
            pltpu.SemaphoreType.DMA,
            pltpu.SemaphoreType.DMA,
            pltpu.SemaphoreType.DMA,
            pltpu.SemaphoreType.DMA,
            pltpu.SemaphoreType.DMA,
            pltpu.SemaphoreType.DMA,
        ],
    )


_R = 2000
_NB = _N // _R


def _tc_first(d0, d1, x, w, dinv_o, z_o):
    deg = d0[...] + d1[...] + 1.0
    dv = lax.rsqrt(deg)
    dinv_o[...] = dv
    z_o[...] = jnp.dot(x[...] * dv, w[...], preferred_element_type=jnp.float32)


def _tc_mid(a0, a1, z, dv, b, w, z_o):
    x = jnp.maximum((a0[...] + a1[...] + z[...]) * dv[...] + b[...], 0.0)
    z_o[...] = jnp.dot(x * dv[...], w[...], preferred_element_type=jnp.float32)


def _tc_last(a0, a1, z, dv, b, bt, fcw, fcb, out, sums_s, cnt_s):
    i = pl.program_id(0)
    x = jnp.maximum((a0[...] + a1[...] + z[...]) * dv[...] + b[...], 0.0)
    brow = bt[0, 0, :]
    gids = lax.broadcasted_iota(jnp.int32, (_G, _R), 0)
    oh = jnp.where(brow[None, :] == gids, 1.0, 0.0)

    @pl.when(i == 0)
    def _():
        sums_s[...] = jnp.zeros_like(sums_s)
        cnt_s[...] = jnp.zeros_like(cnt_s)

    sums_s[...] += jnp.dot(oh, x, preferred_element_type=jnp.float32)
    cnt_s[...] += jnp.broadcast_to(jnp.sum(oh, axis=1, keepdims=True), (_G, _D))

    @pl.when(i == pl.num_programs(0) - 1)
    def _():
        pooled = sums_s[...] / jnp.maximum(cnt_s[...], 1.0)
        out[...] = jnp.dot(pooled, fcw[...],
                           preferred_element_type=jnp.float32) + fcb[...]


def _row_spec(cols):
    return pl.BlockSpec((_R, cols), lambda i: (i, 0))


def _full_spec(r, c):
    return pl.BlockSpec((r, c), lambda i: (0, 0))


_tc_first_call = pl.pallas_call(
    _tc_first,
    grid=(_NB,),
    in_specs=[_row_spec(1), _row_spec(1), _row_spec(_D), _full_spec(_D, _D)],
    out_specs=[_row_spec(1), _row_spec(_D)],
    out_shape=[jax.ShapeDtypeStruct((_N, 1), jnp.float32),
               jax.ShapeDtypeStruct((_N, _D), jnp.float32)],
)

_tc_mid_call = pl.pallas_call(
    _tc_mid,
    grid=(_NB,),
    in_specs=[_row_spec(_D), _row_spec(_D), _row_spec(_D), _row_spec(1),
              _full_spec(1, _D), _full_spec(_D, _D)],
    out_specs=_row_spec(_D),
    out_shape=jax.ShapeDtypeStruct((_N, _D), jnp.float32),
)

_tc_last_call = pl.pallas_call(
    _tc_last,
    grid=(_NB,),
    in_specs=[_row_spec(_D), _row_spec(_D), _row_spec(_D), _row_spec(1),
              _full_spec(1, _D),
              pl.BlockSpec((1, 1, _R), lambda i: (i, 0, 0)),
              _full_spec(_D, 1), _full_spec(1, 1)],
    out_specs=_full_spec(_G, 1),
    out_shape=jax.ShapeDtypeStruct((_G, 1), jnp.float32),
    scratch_shapes=[pltpu.VMEM((_G, _D), jnp.float32),
                    pltpu.VMEM((_G, _D), jnp.float32)],
)


def kernel(edge_index, features, batch, emb, W1, b1, W2, b2, W3, b3, fcW, fcb):
    f32 = jnp.float32
    npad = _EPAD - _E
    pad_ids = jnp.arange(npad, dtype=jnp.int32)
    srcp = jnp.concatenate([edge_index[0], (pad_ids * 997) % _N])
    dstp = jnp.concatenate([edge_index[1], _N + (pad_ids % (_NACC - _N))])
    src3 = srcp.reshape(_NW, _CPW, _CH)
    dst3 = dstp.reshape(_NW, _CPW, _CH)
    featp = jnp.concatenate(
        [features[:, 0], (jnp.arange(_NPAD - _N, dtype=jnp.int32) * 131) % _VOCAB])

    deg_part, x0p = _deg_emb_call()(dst3, featp, emb)
    x0 = x0p[:_N]
    deg0 = deg_part[:_N, None]
    deg1 = deg_part[_NPAD:_NPAD + _N, None]

    dinv, z0 = _tc_first_call(deg0, deg1, x0, W1)

    b1r = b1[None, :]
    b2r = b2[None, :]
    b3r = b3[None, :]
    bt = batch.reshape(_NB, 1, _R)

    s0 = _agg_call()(z0, src3, dst3)
    z1 = _tc_mid_call(s0[0], s0[1], z0, dinv, b1r, W2)
    s1 = _agg_call()(z1, src3, dst3)
    z2 = _tc_mid_call(s1[0], s1[1], z1, dinv, b2r, W3)
    s2 = _agg_call()(z2, src3, dst3)
    out = _tc_last_call(s2[0], s2[1], z2, dinv, b3r, bt,
                        fcW, fcb.reshape(1, 1))
    return out[:, 0].astype(f32)


# speedup vs baseline: 29.2593x; 1.0081x over previous
import functools

import jax
import jax.numpy as jnp
from jax import lax
from jax.experimental import pallas as pl
from jax.experimental.pallas import tpu as pltpu
import jax.experimental.pallas.tpu_sc as plsc

_N = 10000
_E = 320000
_G = 64
_D = 128
_VOCAB = 10000

_NC = 2
_NS = 16
_NW = _NC * _NS

_CH = 128
_CPW = 80
_EPW = _CPW * _CH
_EPAD = _NW * _EPW
_NPW = 320
_NPAD = _NW * _NPW
_FCH = 80
_NACC = 10112
_RPS = _NACC // _NS
_DPS = _NPAD // _NS


def _sc_deg_emb(dst_hbm, feat_hbm, emb_hbm, deg_hbm, x0_hbm,
                didx2, fidx, ones, zb, rows2, dacc, sem, fsem0, fsem1):
    c = lax.axis_index("c")
    s = lax.axis_index("s")
    wid = c * _NS + s

    def _init(i, _):
        ones[pl.ds(i * 16, 16)] = jnp.full((16,), 1.0, jnp.float32)
        zb[pl.ds(i * 16, 16)] = jnp.zeros((16,), jnp.float32)
        return 0
    lax.fori_loop(0, _DPS // 16, _init, 0)
    pltpu.sync_copy(zb, dacc.at[pl.ds(s * _DPS, _DPS)])
    pltpu.sync_copy(dst_hbm.at[wid], didx2)
    plsc.subcore_barrier()

    def _grp(g, _):
        for k in range(8):
            pltpu.async_copy(ones.at[pl.ds(0, _CH)],
                             dacc.at[didx2.at[g * 8 + k]], sem, add=True)
        for k in range(8):
            pltpu.make_async_copy(ones.at[pl.ds(0, _CH)],
                                  dacc.at[didx2.at[g * 8 + k]], sem).wait()
        return 0
    lax.fori_loop(0, _CPW // 8, _grp, 0)

    fbase = wid * _NPW
    fsems = (fsem0, fsem1)
    for j in range(_NPW // _FCH):
        jb = j % 2
        pltpu.sync_copy(feat_hbm.at[pl.ds(fbase + j * _FCH, _FCH)],
                        fidx.at[jb])
        pltpu.async_copy(emb_hbm.at[fidx.at[jb]], rows2.at[jb], fsems[jb])
        if j > 0:
            pb = (j - 1) % 2
            off = fbase + (j - 1) * _FCH
            pltpu.make_async_copy(emb_hbm.at[fidx.at[pb]], rows2.at[pb],
                                  fsems[pb]).wait()
            pltpu.sync_copy(rows2.at[pb], x0_hbm.at[pl.ds(off, _FCH)])
    lastb = (_NPW // _FCH - 1) % 2
    lasto = fbase + (_NPW - _FCH)
    pltpu.make_async_copy(emb_hbm.at[fidx.at[lastb]], rows2.at[lastb],
                          fsems[lastb]).wait()
    pltpu.sync_copy(rows2.at[lastb], x0_hbm.at[pl.ds(lasto, _FCH)])

    plsc.subcore_barrier()
    pltpu.sync_copy(dacc.at[pl.ds(s * _DPS, _DPS)],
                    deg_hbm.at[pl.ds(c * _NPAD + s * _DPS, _DPS)])


@functools.lru_cache(maxsize=None)
def _deg_emb_call():
    mesh = plsc.VectorSubcoreMesh(core_axis_name="c", subcore_axis_name="s",
                                  num_cores=_NC, num_subcores=_NS)
    return pl.kernel(
        _sc_deg_emb,
        out_type=(jax.ShapeDtypeStruct((_NC * _NPAD,), jnp.float32),
                  jax.ShapeDtypeStruct((_NPAD, _D), jnp.float32)),
        mesh=mesh,
        scratch_types=[
            pltpu.VMEM((_CPW, _CH), jnp.int32),
            pltpu.VMEM((2, _FCH), jnp.int32),
            pltpu.VMEM((_DPS,), jnp.float32),
            pltpu.VMEM((_DPS,), jnp.float32),
            pltpu.VMEM((2, _FCH, _D), jnp.float32),
            pltpu.VMEM_SHARED((_NPAD,), jnp.float32),
            pltpu.SemaphoreType.DMA,
            pltpu.SemaphoreType.DMA,
            pltpu.SemaphoreType.DMA,
        ],
    )


def _sc_agg(z_hbm, src_hbm, dst_hbm, out_hbm,
            sidx2, dring, rows, acc,
            gsem0, gsem1, isem0, isem1, isem2, isem3, zsem):
    c = lax.axis_index("c")
    s = lax.axis_index("s")
    wid = c * _NS + s
    rbase = s * _RPS

    gsems = (gsem0, gsem1)
    isems = (isem0, isem1, isem2, isem3)

    def _zr(i, _):
        for j in range(_D // 16):
            rows[1, i, pl.ds(j * 16, 16)] = jnp.zeros((16,), jnp.float32)
        return 0
    lax.fori_loop(0, _CH, _zr, 0)
    pltpu.sync_copy(src_hbm.at[wid], sidx2)
    pltpu.async_copy(z_hbm.at[sidx2.at[0]], rows.at[0], gsem0)
    for k in range(4):
        pltpu.async_copy(dst_hbm.at[wid, k], dring.at[k], isems[k])
    for k in range(4):
        pltpu.async_copy(rows.at[1], acc.at[pl.ds(rbase + k * _CH, _CH)],
                         zsem)
    pltpu.async_copy(rows.at[1, pl.ds(0, _RPS - 4 * _CH)],
                     acc.at[pl.ds(rbase + 4 * _CH, _RPS - 4 * _CH)], zsem)
    for k in range(4):
        pltpu.make_async_copy(rows.at[1], acc.at[pl.ds(rbase + k * _CH, _CH)],
                              zsem).wait()
    pltpu.make_async_copy(rows.at[1, pl.ds(0, _RPS - 4 * _CH)],
                          acc.at[pl.ds(rbase + 4 * _CH, _RPS - 4 * _CH)],
                          zsem).wait()
    plsc.subcore_barrier()

    pltpu.async_copy(z_hbm.at[sidx2.at[1]], rows.at[1], gsem1)

    def _quad(jq, _):
        for b in range(4):
            j = jq * 4 + b
            buf = b % 2
            pltpu.make_async_copy(z_hbm.at[sidx2.at[j]], rows.at[buf],
                                  gsems[buf]).wait()
            pltpu.make_async_copy(dst_hbm.at[wid, j], dring.at[b],
                                  isems[b]).wait()
            pltpu.sync_copy(rows.at[buf], acc.at[dring.at[b]], add=True)
            pltpu.async_copy(dst_hbm.at[wid, j + 4], dring.at[b], isems[b])
            pltpu.async_copy(z_hbm.at[sidx2.at[j + 2]], rows.at[buf],
                             gsems[buf])
        return 0
    lax.fori_loop(0, _CPW // 4 - 1, _quad, 0)
    for b in range(4):
        j = _CPW - 4 + b
        buf = b % 2
        pltpu.make_async_copy(z_hbm.at[sidx2.at[j]], rows.at[buf],
                              gsems[buf]).wait()
        pltpu.make_async_copy(dst_hbm.at[wid, j], dring.at[b],
                              isems[b]).wait()
        pltpu.sync_copy(rows.at[buf], acc.at[dring.at[b]], add=True)
        if j + 2 < _CPW:
            pltpu.async_copy(z_hbm.at[sidx2.at[j + 2]], rows.at[buf],
                             gsems[buf])

    plsc.subcore_barrier()
    nlast = _N - (_NS - 1) * _RPS
    @pl.when(s < _NS - 1)
    def _():
        pltpu.sync_copy(acc.at[pl.ds(rbase, _RPS)],
                        out_hbm.at[c, pl.ds(rbase, _RPS)])
    @pl.when(s == _NS - 1)
    def _():
        pltpu.sync_copy(acc.at[pl.ds(rbase, nlast)],
                        out_hbm.at[c, pl.ds(rbase, nlast)])


@functools.lru_cache(maxsize=None)
def _agg_call():
    mesh = plsc.VectorSubcoreMesh(core_axis_name="c", subcore_axis_name="s",
                                  num_cores=_NC, num_subcores=_NS)
    return pl.kernel(
        _sc_agg,
        out_type=jax.ShapeDtypeStruct((_NC, _N, _D), jnp.float32),
        mesh=mesh,
        scratch_types=[
            pltpu.VMEM((_CPW, _CH), jnp.int32),
            pltpu.VMEM((4, _CH), jnp.int32),
            pltpu.VMEM((2, _CH, _D), jnp.float32),
            pltpu.VMEM_SHARED((_NACC, _D), jnp.float32),
            pltpu.SemaphoreType.DMA,
            pltpu.SemaphoreType.DMA,
            pltpu.SemaphoreType.DMA,
            pltpu.SemaphoreType.DMA,
            pltpu.SemaphoreType.DMA,
            pltpu.SemaphoreType.DMA,
            pltpu.SemaphoreType.DMA,
        ],
    )


_R = 5000
_NB = _N // _R


def _tc_first(d0, d1, x, w, dinv_o, z_o):
    deg = d0[...] + d1[...] + 1.0
    dv = lax.rsqrt(deg)
    dinv_o[...] = dv
    z_o[...] = jnp.dot(x[...] * dv, w[...], preferred_element_type=jnp.float32)


def _tc_mid(a0, a1, z, dv, b, w, z_o):
    x = jnp.maximum((a0[...] + a1[...] + z[...]) * dv[...] + b[...], 0.0)
    z_o[...] = jnp.dot(x * dv[...], w[...], preferred_element_type=jnp.float32)


def _tc_last(a0, a1, z, dv, b, bt, fcw, fcb, out, sums_s, cnt_s):
    i = pl.program_id(0)
    x = jnp.maximum((a0[...] + a1[...] + z[...]) * dv[...] + b[...], 0.0)
    brow = bt[0, 0, :]
    gids = lax.broadcasted_iota(jnp.int32, (_G, _R), 0)
    oh = jnp.where(brow[None, :] == gids, 1.0, 0.0)

    @pl.when(i == 0)
    def _():
        sums_s[...] = jnp.zeros_like(sums_s)
        cnt_s[...] = jnp.zeros_like(cnt_s)

    sums_s[...] += jnp.dot(oh, x, preferred_element_type=jnp.float32)
    cnt_s[...] += jnp.broadcast_to(jnp.sum(oh, axis=1, keepdims=True), (_G, _D))

    @pl.when(i == pl.num_programs(0) - 1)
    def _():
        pooled = sums_s[...] / jnp.maximum(cnt_s[...], 1.0)
        out[...] = jnp.dot(pooled, fcw[...],
                           preferred_element_type=jnp.float32) + fcb[...]


def _row_spec(cols):
    return pl.BlockSpec((_R, cols), lambda i: (i, 0))


def _full_spec(r, c):
    return pl.BlockSpec((r, c), lambda i: (0, 0))


_tc_first_call = pl.pallas_call(
    _tc_first,
    grid=(_NB,),
    in_specs=[_row_spec(1), _row_spec(1), _row_spec(_D), _full_spec(_D, _D)],
    out_specs=[_row_spec(1), _row_spec(_D)],
    out_shape=[jax.ShapeDtypeStruct((_N, 1), jnp.float32),
               jax.ShapeDtypeStruct((_N, _D), jnp.float32)],
)

_tc_mid_call = pl.pallas_call(
    _tc_mid,
    grid=(_NB,),
    in_specs=[_row_spec(_D), _row_spec(_D), _row_spec(_D), _row_spec(1),
              _full_spec(1, _D), _full_spec(_D, _D)],
    out_specs=_row_spec(_D),
    out_shape=jax.ShapeDtypeStruct((_N, _D), jnp.float32),
)

_tc_last_call = pl.pallas_call(
    _tc_last,
    grid=(_NB,),
    in_specs=[_row_spec(_D), _row_spec(_D), _row_spec(_D), _row_spec(1),
              _full_spec(1, _D),
              pl.BlockSpec((1, 1, _R), lambda i: (i, 0, 0)),
              _full_spec(_D, 1), _full_spec(1, 1)],
    out_specs=_full_spec(_G, 1),
    out_shape=jax.ShapeDtypeStruct((_G, 1), jnp.float32),
    scratch_shapes=[pltpu.VMEM((_G, _D), jnp.float32),
                    pltpu.VMEM((_G, _D), jnp.float32)],
)


def kernel(edge_index, features, batch, emb, W1, b1, W2, b2, W3, b3, fcW, fcb):
    f32 = jnp.float32
    npad = _EPAD - _E
    pad_ids = jnp.arange(npad, dtype=jnp.int32)
    srcp = jnp.concatenate([edge_index[0], (pad_ids * 997) % _N])
    dstp = jnp.concatenate([edge_index[1], _N + (pad_ids % (_NACC - _N))])
    src3 = srcp.reshape(_NW, _CPW, _CH)
    dst3 = dstp.reshape(_NW, _CPW, _CH)
    featp = jnp.concatenate(
        [features[:, 0], (jnp.arange(_NPAD - _N, dtype=jnp.int32) * 131) % _VOCAB])

    deg_part, x0p = _deg_emb_call()(dst3, featp, emb)
    x0 = x0p[:_N]
    deg0 = deg_part[:_N, None]
    deg1 = deg_part[_NPAD:_NPAD + _N, None]

    dinv, z0 = _tc_first_call(deg0, deg1, x0, W1)

    b1r = b1[None, :]
    b2r = b2[None, :]
    b3r = b3[None, :]
    bt = batch.reshape(_NB, 1, _R)

    s0 = _agg_call()(z0, src3, dst3)
    z1 = _tc_mid_call(s0[0], s0[1], z0, dinv, b1r, W2)
    s1 = _agg_call()(z1, src3, dst3)
    z2 = _tc_mid_call(s1[0], s1[1], z1, dinv, b2r, W3)
    s2 = _agg_call()(z2, src3, dst3)
    out = _tc_last_call(s2[0], s2[1], z2, dinv, b3r, bt,
                        fcW, fcb.reshape(1, 1))
    return out[:, 0].astype(f32)


# speedup vs baseline: 31.5622x; 1.0787x over previous
import functools

import jax
import jax.numpy as jnp
from jax import lax
from jax.experimental import pallas as pl
from jax.experimental.pallas import tpu as pltpu
import jax.experimental.pallas.tpu_sc as plsc

_N = 10000
_E = 320000
_G = 64
_D = 128
_VOCAB = 10000

_NC = 2
_NS = 16
_NW = _NC * _NS

_CH = 128
_CPW = 81
_EPW = _CPW * _CH
_EPAD = _NW * _EPW
_NPW = 320
_NPAD = _NW * _NPW
_FCH = 80
_NACC = 10016
_RPS = 632
_DPS = _NPAD // _NS


def _sc_deg_emb(dst_hbm, feat_hbm, emb_hbm, deg_hbm, x0_hbm,
                didx2, fidx, ones, zb, rows2, dacc, sem, fsem0, fsem1):
    c = lax.axis_index("c")
    s = lax.axis_index("s")
    wid = c * _NS + s

    def _init(i, _):
        ones[pl.ds(i * 16, 16)] = jnp.full((16,), 1.0, jnp.float32)
        zb[pl.ds(i * 16, 16)] = jnp.zeros((16,), jnp.float32)
        return 0
    lax.fori_loop(0, _DPS // 16, _init, 0)
    pltpu.sync_copy(zb, dacc.at[pl.ds(s * _DPS, _DPS)])
    plsc.subcore_barrier()

    ebase = wid * _EPW
    def _grp(g, _):
        for k in range(8):
            pltpu.async_copy(
                dst_hbm.at[pl.ds(ebase + (g * 8 + k) * _CH, _CH)],
                didx2.at[k], fsem0)
        for k in range(8):
            pltpu.make_async_copy(
                dst_hbm.at[pl.ds(ebase + (g * 8 + k) * _CH, _CH)],
                didx2.at[k], fsem0).wait()
        for k in range(8):
            pltpu.async_copy(ones.at[pl.ds(0, _CH)],
                             dacc.at[didx2.at[k]], sem, add=True)
        for k in range(8):
            pltpu.make_async_copy(ones.at[pl.ds(0, _CH)],
                                  dacc.at[didx2.at[k]], sem).wait()
        return 0
    lax.fori_loop(0, _CPW // 8, _grp, 0)
    for k in range(_CPW - 8 * (_CPW // 8)):
        pltpu.sync_copy(
            dst_hbm.at[pl.ds(ebase + (8 * (_CPW // 8) + k) * _CH, _CH)],
            didx2.at[k])
        pltpu.sync_copy(ones.at[pl.ds(0, _CH)],
                        dacc.at[didx2.at[k]], add=True)

    fbase = wid * _NPW
    fsems = (fsem0, fsem1)
    for j in range(_NPW // _FCH):
        jb = j % 2
        pltpu.sync_copy(feat_hbm.at[pl.ds(fbase + j * _FCH, _FCH)],
                        fidx.at[jb])
        pltpu.async_copy(emb_hbm.at[fidx.at[jb]], rows2.at[jb], fsems[jb])
        if j > 0:
            pb = (j - 1) % 2
            off = fbase + (j - 1) * _FCH
            pltpu.make_async_copy(emb_hbm.at[fidx.at[pb]], rows2.at[pb],
                                  fsems[pb]).wait()
            pltpu.sync_copy(rows2.at[pb], x0_hbm.at[pl.ds(off, _FCH)])
    lastb = (_NPW // _FCH - 1) % 2
    lasto = fbase + (_NPW - _FCH)
    pltpu.make_async_copy(emb_hbm.at[fidx.at[lastb]], rows2.at[lastb],
                          fsems[lastb]).wait()
    pltpu.sync_copy(rows2.at[lastb], x0_hbm.at[pl.ds(lasto, _FCH)])

    plsc.subcore_barrier()
    pltpu.sync_copy(dacc.at[pl.ds(s * _DPS, _DPS)],
                    deg_hbm.at[pl.ds(c * _NPAD + s * _DPS, _DPS)])


@functools.lru_cache(maxsize=None)
def _deg_emb_call():
    mesh = plsc.VectorSubcoreMesh(core_axis_name="c", subcore_axis_name="s",
                                  num_cores=_NC, num_subcores=_NS)
    return pl.kernel(
        _sc_deg_emb,
        out_type=(jax.ShapeDtypeStruct((_NC * _NPAD,), jnp.float32),
                  jax.ShapeDtypeStruct((_NPAD, _D), jnp.float32)),
        mesh=mesh,
        scratch_types=[
            pltpu.VMEM((8, _CH), jnp.int32),
            pltpu.VMEM((2, _FCH), jnp.int32),
            pltpu.VMEM((_DPS,), jnp.float32),
            pltpu.VMEM((_DPS,), jnp.float32),
            pltpu.VMEM((2, _FCH, _D), jnp.float32),
            pltpu.VMEM_SHARED((_NPAD,), jnp.float32),
            pltpu.SemaphoreType.DMA,
            pltpu.SemaphoreType.DMA,
            pltpu.SemaphoreType.DMA,
        ],
    )


def _sc_agg(z_hbm, src_hbm, dst_hbm, out_hbm,
            ring, rows, acc,
            g0, g1, g2, s0, s1, s2, p0, p1, p2, q0, q1, q2):
    c = lax.axis_index("c")
    s = lax.axis_index("s")
    wid = c * _NS + s
    ebase = wid * _EPW
    rbase = s * _RPS
    gs = (g0, g1, g2)
    ss = (s0, s1, s2)
    ps = (p0, p1, p2)
    qs = (q0, q1, q2)

    def _zr(i, _):
        for j in range(_D // 16):
            rows[2, i, pl.ds(j * 16, 16)] = jnp.zeros((16,), jnp.float32)
        return 0
    lax.fori_loop(0, _CH, _zr, 0)
    for k in range(3):
        pltpu.async_copy(src_hbm.at[pl.ds(ebase + k * _CH, _CH)],
                         ring.at[k], ps[k])
    for k in range(2):
        pltpu.async_copy(dst_hbm.at[pl.ds(ebase + k * _CH, _CH)],
                         ring.at[3 + k], qs[k])
    for k in range(4):
        pltpu.async_copy(rows.at[2], acc.at[pl.ds(rbase + k * _CH, _CH)], q2)
    @pl.when(s < _NS - 1)
    def _():
        pltpu.async_copy(rows.at[2, pl.ds(0, _RPS - 4 * _CH)],
                         acc.at[pl.ds(rbase + 4 * _CH, _RPS - 4 * _CH)], q2)
        pltpu.make_async_copy(rows.at[2, pl.ds(0, _RPS - 4 * _CH)],
                              acc.at[pl.ds(rbase + 4 * _CH, _RPS - 4 * _CH)],
                              q2).wait()
    @pl.when(s == _NS - 1)
    def _():
        zl = _NACC - 15 * _RPS - 4 * _CH
        pltpu.async_copy(rows.at[2, pl.ds(0, zl)],
                         acc.at[pl.ds(rbase + 4 * _CH, zl)], q2)
        pltpu.make_async_copy(rows.at[2, pl.ds(0, zl)],
                              acc.at[pl.ds(rbase + 4 * _CH, zl)], q2).wait()
    for k in range(4):
        pltpu.make_async_copy(rows.at[2], acc.at[pl.ds(rbase + k * _CH, _CH)],
                              q2).wait()
    pltpu.make_async_copy(src_hbm.at[pl.ds(ebase, _CH)],
                          ring.at[0], ps[0]).wait()
    pltpu.async_copy(z_hbm.at[ring.at[0]], rows.at[0], gs[0])
    pltpu.make_async_copy(src_hbm.at[pl.ds(ebase + _CH, _CH)],
                          ring.at[1], ps[1]).wait()
    pltpu.async_copy(z_hbm.at[ring.at[1]], rows.at[1], gs[1])
    plsc.subcore_barrier()

    def _chunk(j, b, first, src_fetch, dst_fetch, gather):
        b2 = (b + 2) % 3
        pltpu.make_async_copy(z_hbm.at[ring.at[b]], rows.at[b], gs[b]).wait()
        if src_fetch:
            pltpu.async_copy(src_hbm.at[pl.ds(ebase + (j + 3) * _CH, _CH)],
                             ring.at[b], ps[b])
        if not first:
            pltpu.make_async_copy(rows.at[b2], acc.at[ring.at[3 + b2]],
                                  ss[b2]).wait()
        if dst_fetch:
            pltpu.async_copy(dst_hbm.at[pl.ds(ebase + (j + 2) * _CH, _CH)],
                             ring.at[3 + b2], qs[b2])
        if gather:
            pltpu.make_async_copy(src_hbm.at[pl.ds(ebase + (j + 2) * _CH, _CH)],
                                  ring.at[b2], ps[b2]).wait()
            pltpu.async_copy(z_hbm.at[ring.at[b2]], rows.at[b2], gs[b2])
        pltpu.make_async_copy(dst_hbm.at[pl.ds(ebase + j * _CH, _CH)],
                              ring.at[3 + b], qs[b]).wait()
        pltpu.async_copy(rows.at[b], acc.at[ring.at[3 + b]], ss[b], add=True)

    _chunk(0, 0, True, True, True, True)
    _chunk(1, 1, False, True, True, True)
    _chunk(2, 2, False, True, True, True)

    def _triple(jq, _):
        for b in range(3):
            j = 3 + jq * 3 + b
            _chunk(j, b, False, True, True, True)
        return 0
    lax.fori_loop(0, (_CPW - 6) // 3, _triple, 0)
    _chunk(_CPW - 3, 0, False, False, True, True)
    _chunk(_CPW - 2, 1, False, False, False, False)
    _chunk(_CPW - 1, 2, False, False, False, False)
    pltpu.make_async_copy(rows.at[2], acc.at[ring.at[5]], ss[2]).wait()

    plsc.subcore_barrier()
    nlast = _N - (_NS - 1) * _RPS
    @pl.when(s < _NS - 1)
    def _():
        pltpu.sync_copy(acc.at[pl.ds(rbase, _RPS)],
                        out_hbm.at[c, pl.ds(rbase, _RPS)])
    @pl.when(s == _NS - 1)
    def _():
        pltpu.sync_copy(acc.at[pl.ds(rbase, nlast)],
                        out_hbm.at[c, pl.ds(rbase, nlast)])


@functools.lru_cache(maxsize=None)
def _agg_call():
    mesh = plsc.VectorSubcoreMesh(core_axis_name="c", subcore_axis_name="s",
                                  num_cores=_NC, num_subcores=_NS)
    return pl.kernel(
        _sc_agg,
        out_type=jax.ShapeDtypeStruct((_NC, _N, _D), jnp.float32),
        mesh=mesh,
        scratch_types=[
            pltpu.VMEM((6, _CH), jnp.int32),
            pltpu.VMEM((3, _CH, _D), jnp.float32),
            pltpu.VMEM_SHARED((_NACC, _D), jnp.float32),
        ] + [pltpu.SemaphoreType.DMA] * 12,
    )


_R = 5000
_NB = _N // _R


def _tc_first(d0, d1, x, w, dinv_o, z_o):
    deg = d0[...] + d1[...] + 1.0
    dv = lax.rsqrt(deg)
    dinv_o[...] = dv
    z_o[...] = jnp.dot(x[...] * dv, w[...], preferred_element_type=jnp.float32)


def _tc_mid(a0, a1, z, dv, b, w, z_o):
    x = jnp.maximum((a0[...] + a1[...] + z[...]) * dv[...] + b[...], 0.0)
    z_o[...] = jnp.dot(x * dv[...], w[...], preferred_element_type=jnp.float32)


def _tc_last(a0, a1, z, dv, b, bt, fcw, fcb, out, sums_s, cnt_s):
    i = pl.program_id(0)
    x = jnp.maximum((a0[...] + a1[...] + z[...]) * dv[...] + b[...], 0.0)
    brow = bt[0, 0, :]
    gids = lax.broadcasted_iota(jnp.int32, (_G, _R), 0)
    oh = jnp.where(brow[None, :] == gids, 1.0, 0.0)

    @pl.when(i == 0)
    def _():
        sums_s[...] = jnp.zeros_like(sums_s)
        cnt_s[...] = jnp.zeros_like(cnt_s)

    sums_s[...] += jnp.dot(oh, x, preferred_element_type=jnp.float32)
    cnt_s[...] += jnp.broadcast_to(jnp.sum(oh, axis=1, keepdims=True), (_G, _D))

    @pl.when(i == pl.num_programs(0) - 1)
    def _():
        pooled = sums_s[...] / jnp.maximum(cnt_s[...], 1.0)
        out[...] = jnp.dot(pooled, fcw[...],
                           preferred_element_type=jnp.float32) + fcb[...]


def _row_spec(cols):
    return pl.BlockSpec((_R, cols), lambda i: (i, 0))


def _full_spec(r, c):
    return pl.BlockSpec((r, c), lambda i: (0, 0))


_tc_first_call = pl.pallas_call(
    _tc_first,
    grid=(_NB,),
    in_specs=[_row_spec(1), _row_spec(1), _row_spec(_D), _full_spec(_D, _D)],
    out_specs=[_row_spec(1), _row_spec(_D)],
    out_shape=[jax.ShapeDtypeStruct((_N, 1), jnp.float32),
               jax.ShapeDtypeStruct((_N, _D), jnp.float32)],
)

_tc_mid_call = pl.pallas_call(
    _tc_mid,
    grid=(_NB,),
    in_specs=[_row_spec(_D), _row_spec(_D), _row_spec(_D), _row_spec(1),
              _full_spec(1, _D), _full_spec(_D, _D)],
    out_specs=_row_spec(_D),
    out_shape=jax.ShapeDtypeStruct((_N, _D), jnp.float32),
)

_tc_last_call = pl.pallas_call(
    _tc_last,
    grid=(_NB,),
    in_specs=[_row_spec(_D), _row_spec(_D), _row_spec(_D), _row_spec(1),
              _full_spec(1, _D),
              pl.BlockSpec((1, 1, _R), lambda i: (i, 0, 0)),
              _full_spec(_D, 1), _full_spec(1, 1)],
    out_specs=_full_spec(_G, 1),
    out_shape=jax.ShapeDtypeStruct((_G, 1), jnp.float32),
    scratch_shapes=[pltpu.VMEM((_G, _D), jnp.float32),
                    pltpu.VMEM((_G, _D), jnp.float32)],
)


def kernel(edge_index, features, batch, emb, W1, b1, W2, b2, W3, b3, fcW, fcb):
    f32 = jnp.float32
    npad = _EPAD - _E
    pad_ids = jnp.arange(npad, dtype=jnp.int32)
    srcp = jnp.concatenate([edge_index[0], (pad_ids * 997) % _N])
    dstp = jnp.concatenate([edge_index[1], _N + (pad_ids % (_NACC - _N))])

    featp = jnp.concatenate(
        [features[:, 0], (jnp.arange(_NPAD - _N, dtype=jnp.int32) * 131) % _VOCAB])

    deg_part, x0p = _deg_emb_call()(dstp, featp, emb)
    x0 = x0p[:_N]
    deg0 = deg_part[:_N, None]
    deg1 = deg_part[_NPAD:_NPAD + _N, None]

    dinv, z0 = _tc_first_call(deg0, deg1, x0, W1)

    b1r = b1[None, :]
    b2r = b2[None, :]
    b3r = b3[None, :]
    bt = batch.reshape(_NB, 1, _R)

    s0 = _agg_call()(z0, srcp, dstp)
    z1 = _tc_mid_call(s0[0], s0[1], z0, dinv, b1r, W2)
    s1 = _agg_call()(z1, srcp, dstp)
    z2 = _tc_mid_call(s1[0], s1[1], z1, dinv, b2r, W3)
    s2 = _agg_call()(z2, srcp, dstp)
    out = _tc_last_call(s2[0], s2[1], z2, dinv, b3r, bt,
                        fcW, fcb.reshape(1, 1))
    return out[:, 0].astype(f32)
